# baseline (speedup 1.0000x reference)
import functools

import jax
import jax.numpy as jnp
from jax import lax
from jax.experimental import pallas as pl
from jax.experimental.pallas import tpu as pltpu
from jax.experimental.pallas import tpu_sc as plsc

N_EMB = 8192
DIM = 32
N_ROWS = 4096
BM = 256
GRID = N_ROWS // BM

NC = 2
NS = 16
NW = NC * NS
BPW = N_ROWS // NW
HPW = N_EMB // NS


def _argmin_body(z_ref, w_ref, idx_ref, znat_ref, wd_ref, idx_acc):
    pid = pl.program_id(0)
    zb = z_ref[0].reshape(DIM, BM)
    znat_ref[...] = zb.reshape(1, DIM, BM)
    wd_ref[...] = w_ref[pl.ds(pid * (N_EMB // GRID), N_EMB // GRID), :]
    w = w_ref[...]
    zz = jnp.sum(zb * zb, axis=0, keepdims=True)
    wsq = jnp.sum(w * w, axis=1, keepdims=True)
    m2 = jax.lax.dot_general(w + w, zb, (((1,), (0,)), ((), ())),
                             preferred_element_type=jnp.float32)
    def _split(x):
        n = x.shape[0]
        x3 = x.reshape(n // 16, 16, BM)
        return (x3[:, :8, :].reshape(n // 2, BM),
                x3[:, 8:, :].reshape(n // 2, BM))

    d = (zz - m2) + wsq
    a, b = _split(d)
    i = jnp.where(b < a, jnp.int32(8), jnp.int32(0))
    v = jnp.minimum(a, b)
    step = 16
    while v.shape[0] > 8:
        a, b = _split(v)
        ia, ib = _split(i)
        i = jnp.where(b < a, ib + jnp.int32(step), ia)
        v = jnp.minimum(a, b)
        step *= 2
    vmin = jnp.min(v, axis=0, keepdims=True)
    rows = lax.broadcasted_iota(jnp.int32, (8, BM), 0)
    idx = jnp.min(jnp.where(v == vmin, i + rows, jnp.int32(2**30)), axis=0)
    idx_acc[pl.ds(pid * (BM // 128), BM // 128), :] = idx.reshape(BM // 128, 128)
    idx_ref[...] = idx_acc[...]


def _argmin_call(z_e, w):
    blocks_per_batch = 1024 // BM
    rows_per_block = BM // 32
    return pl.pallas_call(
        _argmin_body,
        grid=(GRID,),
        in_specs=[
            pl.BlockSpec((1, DIM, rows_per_block, 32),
                         lambda i: (i // blocks_per_batch, 0,
                                    i % blocks_per_batch, 0)),
            pl.BlockSpec((N_EMB, DIM), lambda i: (0, 0)),
        ],
        out_specs=[
            pl.BlockSpec((N_ROWS // 128, 128), lambda i: (0, 0)),
            pl.BlockSpec((1, DIM, BM),
                         lambda i: (i // blocks_per_batch, 0,
                                    i % blocks_per_batch)),
            pl.BlockSpec((N_EMB // GRID, DIM), lambda i: (i, 0)),
        ],
        out_shape=[
            jax.ShapeDtypeStruct((N_ROWS // 128, 128), jnp.int32),
            jax.ShapeDtypeStruct((4, DIM, 1024), jnp.float32),
            jax.ShapeDtypeStruct((N_EMB, DIM), jnp.float32),
        ],
        scratch_shapes=[pltpu.VMEM((N_ROWS // 128, 128), jnp.int32)],
    )(z_e, w)


def _sc_body(w_hbm, idx_hbm, z_hbm, qst_hbm, counts_hbm, lp_hbm,
             idx_v, rows_v, ztile_v, stt_v, ones_v, zeros_v, acc_v,
             hist_sh, loss_sh, sem):
    c = lax.axis_index("c")
    s = lax.axis_index("s")
    wid = s * NC + c
    base = wid * BPW
    bb = base // 1024
    hw0 = base % 1024

    pltpu.sync_copy(idx_hbm.at[pl.ds(base, BPW)], idx_v)
    pltpu.async_copy(w_hbm.at[idx_v], rows_v, sem).wait()
    pltpu.sync_copy(z_hbm.at[bb, :, pl.ds(hw0, BPW)], ztile_v)

    lane = lax.iota(jnp.int32, 16)

    def _st_step(k, acc):
        ch = k >> 3
        g = k & 7
        qv = plsc.load_gather(rows_v, [g * 16 + lane,
                                       jnp.full((16,), ch, jnp.int32)])
        zv = ztile_v[ch, pl.ds(g * 16, 16)]
        stt_v[ch, pl.ds(g * 16, 16)] = zv + (qv - zv)
        dv = zv - qv
        return acc + dv * dv

    acc = lax.fori_loop(0, (BPW * DIM) // 16, _st_step,
                        jnp.zeros((16,), jnp.float32))
    acc_v[...] = acc
    pltpu.sync_copy(stt_v, qst_hbm.at[bb, :, pl.ds(hw0, BPW)])
    pltpu.sync_copy(acc_v, loss_sh.at[s])

    def _fill_zeros(i, _):
        zeros_v[pl.ds(i * 16, 16)] = jnp.zeros((16,), jnp.float32)
        return _
    lax.fori_loop(0, HPW // 16, _fill_zeros, None)

    def _fill_ones(i, _):
        ones_v[pl.ds(i * 16, 16)] = jnp.full((16,), 1.0, jnp.float32)
        return _
    lax.fori_loop(0, BPW // 16, _fill_ones, None)
    pltpu.sync_copy(zeros_v, hist_sh.at[pl.ds(s * HPW, HPW)])
    plsc.subcore_barrier()
    pltpu.sync_copy(ones_v, hist_sh.at[idx_v], add=True)
    plsc.subcore_barrier()

    @pl.when(s == 0)
    def _():
        pltpu.sync_copy(hist_sh, counts_hbm.at[c])
        pltpu.sync_copy(loss_sh, lp_hbm.at[c])


def _sc_call(W, idx, z_nat):
    mesh = plsc.VectorSubcoreMesh(core_axis_name="c", subcore_axis_name="s")
    f = pl.kernel(
        _sc_body,
        out_type=[
            jax.ShapeDtypeStruct((4, DIM, 1024), jnp.float32),
            jax.ShapeDtypeStruct((NC, N_EMB), jnp.float32),
            jax.ShapeDtypeStruct((NC, NS, 16), jnp.float32),
        ],
        mesh=mesh,
        scratch_types=[
            pltpu.VMEM((BPW,), jnp.int32),
            pltpu.VMEM((BPW, DIM), jnp.float32),
            pltpu.VMEM((DIM, BPW), jnp.float32),
            pltpu.VMEM((DIM, BPW), jnp.float32),
            pltpu.VMEM((BPW,), jnp.float32),
            pltpu.VMEM((HPW,), jnp.float32),
            pltpu.VMEM((16,), jnp.float32),
            pltpu.VMEM_SHARED((N_EMB,), jnp.float32),
            pltpu.VMEM_SHARED((NS, 16), jnp.float32),
            pltpu.SemaphoreType.DMA,
        ],
        compiler_params=pltpu.CompilerParams(use_tc_tiling_on_sc=False, needs_layout_passes=False),
    )
    return f(W, idx, z_nat)


def _fin_body(c_ref, lp_ref, loss_ref, perp_ref):
    msq = jnp.sum(lp_ref[...]) * (1.0 / (N_ROWS * DIM))
    loss_ref[...] = (0.25 * msq + msq).reshape(1, 1)
    cc = c_ref[0:1, :] + c_ref[1:2, :]
    p = cc * (1.0 / N_ROWS)
    ent = p * jnp.log(p + 1e-10)
    perp_ref[...] = jnp.exp(-jnp.sum(ent)).reshape(1, 1)


def _fin_call(counts2, lp):
    return pl.pallas_call(
        _fin_body,
        out_shape=[
            jax.ShapeDtypeStruct((1, 1), jnp.float32),
            jax.ShapeDtypeStruct((1, 1), jnp.float32),
        ],
    )(counts2, lp)


def kernel(z_e, W):
    b, c, h, w = z_e.shape
    idx2, z_nat, w_dense = _argmin_call(z_e, W)
    encoding_indices = idx2.reshape(-1)
    qst_nat, counts2, lp = _sc_call(w_dense, encoding_indices, z_nat)
    vq_loss, perplexity = _fin_call(counts2, lp.reshape(NC * NS, 16))
    z_q_st = qst_nat.reshape(b, c, h, w)
    indices = encoding_indices.reshape(b, h, w)
    return (z_q_st, vq_loss[0, 0], perplexity[0, 0], indices)


# speedup vs baseline: 1.0355x; 1.0355x over previous
import functools

import jax
import jax.numpy as jnp
from jax import lax
from jax.experimental import pallas as pl
from jax.experimental.pallas import tpu as pltpu
from jax.experimental.pallas import tpu_sc as plsc

N_EMB = 8192
DIM = 32
N_ROWS = 4096
BM = 256
GRID = N_ROWS // BM

NC = 2
NS = 16
NW = NC * NS
BPW = N_ROWS // NW
HPW = N_EMB // NS


def _argmin_body(z_ref, w_ref, idx_ref, znat_ref):
    zb = z_ref[0].reshape(DIM, BM)
    znat_ref[...] = zb.reshape(1, DIM, BM)
    w = w_ref[...]
    zz = jnp.sum(zb * zb, axis=0, keepdims=True)
    wsq = jnp.sum(w * w, axis=1, keepdims=True)
    m2 = jax.lax.dot_general(w + w, zb, (((1,), (0,)), ((), ())),
                             preferred_element_type=jnp.float32)
    def _split(x):
        n = x.shape[0]
        x3 = x.reshape(n // 16, 16, BM)
        return (x3[:, :8, :].reshape(n // 2, BM),
                x3[:, 8:, :].reshape(n // 2, BM))

    d = (zz - m2) + wsq
    a, b = _split(d)
    i = jnp.where(b < a, jnp.int32(8), jnp.int32(0))
    v = jnp.minimum(a, b)
    step = 16
    while v.shape[0] > 8:
        a, b = _split(v)
        ia, ib = _split(i)
        i = jnp.where(b < a, ib + jnp.int32(step), ia)
        v = jnp.minimum(a, b)
        step *= 2
    vmin = jnp.min(v, axis=0, keepdims=True)
    rows = lax.broadcasted_iota(jnp.int32, (8, BM), 0)
    idx = jnp.min(jnp.where(v == vmin, i + rows, jnp.int32(2**30)), axis=0)
    idx_ref[...] = idx.reshape(1, 1, BM)


def _argmin_call(z_e, w):
    blocks_per_batch = 1024 // BM
    rows_per_block = BM // 32
    return pl.pallas_call(
        _argmin_body,
        grid=(GRID,),
        in_specs=[
            pl.BlockSpec((1, DIM, rows_per_block, 32),
                         lambda i: (i // blocks_per_batch, 0,
                                    i % blocks_per_batch, 0)),
            pl.BlockSpec((N_EMB, DIM), lambda i: (0, 0)),
        ],
        out_specs=[
            pl.BlockSpec((1, 1, BM), lambda i: (i, 0, 0)),
            pl.BlockSpec((1, DIM, BM),
                         lambda i: (i // blocks_per_batch, 0,
                                    i % blocks_per_batch)),
        ],
        out_shape=[
            jax.ShapeDtypeStruct((GRID, 1, BM), jnp.int32),
            jax.ShapeDtypeStruct((4, DIM, 1024), jnp.float32),
        ],
    )(z_e, w)


def _sc_body(w_hbm, idx_hbm, z_hbm, qst_hbm, counts_hbm, lp_hbm,
             idx_v, rows_v, ztile_v, stt_v, ones_v, zeros_v, acc_v,
             hist_sh, loss_sh, sem):
    c = lax.axis_index("c")
    s = lax.axis_index("s")
    wid = s * NC + c
    base = wid * BPW
    bb = base // 1024
    hw0 = base % 1024

    pltpu.sync_copy(idx_hbm.at[pl.ds(base, BPW)], idx_v)
    pltpu.async_copy(w_hbm.at[idx_v], rows_v, sem).wait()
    pltpu.sync_copy(z_hbm.at[bb, :, pl.ds(hw0, BPW)], ztile_v)

    lane = lax.iota(jnp.int32, 16)

    def _st_step(k, acc):
        ch = k >> 3
        g = k & 7
        qv = plsc.load_gather(rows_v, [g * 16 + lane,
                                       jnp.full((16,), ch, jnp.int32)])
        zv = ztile_v[ch, pl.ds(g * 16, 16)]
        stt_v[ch, pl.ds(g * 16, 16)] = zv + (qv - zv)
        dv = zv - qv
        return acc + dv * dv

    acc = lax.fori_loop(0, (BPW * DIM) // 16, _st_step,
                        jnp.zeros((16,), jnp.float32))
    acc_v[...] = acc
    pltpu.sync_copy(stt_v, qst_hbm.at[bb, :, pl.ds(hw0, BPW)])
    pltpu.sync_copy(acc_v, loss_sh.at[s])

    def _fill_zeros(i, _):
        zeros_v[pl.ds(i * 16, 16)] = jnp.zeros((16,), jnp.float32)
        return _
    lax.fori_loop(0, HPW // 16, _fill_zeros, None)

    def _fill_ones(i, _):
        ones_v[pl.ds(i * 16, 16)] = jnp.full((16,), 1.0, jnp.float32)
        return _
    lax.fori_loop(0, BPW // 16, _fill_ones, None)
    pltpu.sync_copy(zeros_v, hist_sh.at[pl.ds(s * HPW, HPW)])
    plsc.subcore_barrier()
    pltpu.sync_copy(ones_v, hist_sh.at[idx_v], add=True)
    plsc.subcore_barrier()

    @pl.when(s == 0)
    def _():
        pltpu.sync_copy(hist_sh, counts_hbm.at[c])
        pltpu.sync_copy(loss_sh, lp_hbm.at[c])


def _sc_call(W, idx, z_nat):
    mesh = plsc.VectorSubcoreMesh(core_axis_name="c", subcore_axis_name="s")
    f = pl.kernel(
        _sc_body,
        out_type=[
            jax.ShapeDtypeStruct((4, DIM, 1024), jnp.float32),
            jax.ShapeDtypeStruct((NC, N_EMB), jnp.float32),
            jax.ShapeDtypeStruct((NC, NS, 16), jnp.float32),
        ],
        mesh=mesh,
        scratch_types=[
            pltpu.VMEM((BPW,), jnp.int32),
            pltpu.VMEM((BPW, DIM), jnp.float32),
            pltpu.VMEM((DIM, BPW), jnp.float32),
            pltpu.VMEM((DIM, BPW), jnp.float32),
            pltpu.VMEM((BPW,), jnp.float32),
            pltpu.VMEM((HPW,), jnp.float32),
            pltpu.VMEM((16,), jnp.float32),
            pltpu.VMEM_SHARED((N_EMB,), jnp.float32),
            pltpu.VMEM_SHARED((NS, 16), jnp.float32),
            pltpu.SemaphoreType.DMA,
        ],
        compiler_params=pltpu.CompilerParams(use_tc_tiling_on_sc=False, needs_layout_passes=False),
    )
    return f(W, idx, z_nat)


def _fin_body(c_ref, lp_ref, loss_ref, perp_ref):
    msq = jnp.sum(lp_ref[...]) * (1.0 / (N_ROWS * DIM))
    loss_ref[...] = (0.25 * msq + msq).reshape(1, 1)
    cc = c_ref[0:1, :] + c_ref[1:2, :]
    p = cc * (1.0 / N_ROWS)
    ent = p * jnp.log(p + 1e-10)
    perp_ref[...] = jnp.exp(-jnp.sum(ent)).reshape(1, 1)


def _fin_call(counts2, lp):
    return pl.pallas_call(
        _fin_body,
        out_shape=[
            jax.ShapeDtypeStruct((1, 1), jnp.float32),
            jax.ShapeDtypeStruct((1, 1), jnp.float32),
        ],
    )(counts2, lp)


def kernel(z_e, W):
    b, c, h, w = z_e.shape
    idx3, z_nat = _argmin_call(z_e, W)
    encoding_indices = idx3.reshape(-1)
    qst_nat, counts2, lp = _sc_call(W, encoding_indices, z_nat)
    vq_loss, perplexity = _fin_call(counts2, lp.reshape(NC * NS, 16))
    z_q_st = qst_nat.reshape(b, c, h, w)
    indices = encoding_indices.reshape(b, h, w)
    return (z_q_st, vq_loss[0, 0], perplexity[0, 0], indices)
